# baseline (speedup 1.0000x reference)
import jax
import jax.numpy as jnp
from jax import lax
from jax.experimental import pallas as pl
from jax.experimental.pallas import tpu as pltpu
from jax.experimental.pallas import tpu_sc as plsc

_NUM_RELATION = 7
_MAX_SEQ_DIST = 10
_B = 64
_NN = 50000
_NE = 1600000
_NC = 2
_NS = 16
_W = _NC * _NS
_EPW = _NE // _W
_K1 = 2000
_NW1 = _EPW // _K1
_K2 = 2000
_NW2 = _EPW // _K2
_FDIM = 25
_RSQRT_MAGIC = 0x5F3759DF
_CH = 80
_DC = 256
_SPC = _NS * _EPW + 64 * 16 + 32

_PARAMS = pltpu.CompilerParams(needs_layout_passes=False)


def _mesh():
    return plsc.VectorSubcoreMesh(
        core_axis_name="c", subcore_axis_name="s",
        num_cores=_NC, num_subcores=_NS)


def _rank_body(ni_hbm, no_hbm, r_hbm, a2r_hbm, n2g_hbm,
               ta_hbm, tb_hbm, nedge_hbm, nstart_hbm,
               ta_sp, tb_sp, n2g_sp, a2r_sp, hist_sp,
               histv, cur, stage,
               lcnt_t, gst_t, s_t, shift_t, tmp16, idx16, dbuf, dbuf2,
               ni_b, no_b, r_b, ta_b, tb_b, g_b, ri_b, ro_b, dest_b,
               sem, sem2):
    c = lax.axis_index("c")
    s = lax.axis_index("s")
    wid = c * _NS + s

    @pl.when(s == 0)
    def _():
        pltpu.sync_copy(n2g_hbm, n2g_sp)
        pltpu.sync_copy(a2r_hbm, a2r_sp)
    plsc.subcore_barrier()

    iota = lax.iota(jnp.int32, 16)
    zeros = jnp.zeros((16,), jnp.int32)
    ones = jnp.ones((16,), jnp.int32)
    lane64 = iota * 64

    for widx_off in (0, _NS):
        widx = s + widx_off

        def _z(i, carry):
            ta_b[pl.ds(i * 16, 16)] = zeros
            return carry
        lax.fori_loop(0, 64, _z, 0)

        def _hin(t, buf):
            pltpu.async_copy(ni_hbm.at[pl.ds(widx * _EPW + t * _K2, _K2)],
                             buf, sem)

        _hin(0, ni_b)

        def _hwin(t, carry):
            pltpu.make_async_copy(ni_hbm.at[pl.ds(0, _K2)], ni_b, sem).wait()

            @pl.when(((t + 1) < _NW2) & ((t & 1) == 0))
            def _():
                _hin(t + 1, no_b)

            @pl.when(((t + 1) < _NW2) & ((t & 1) == 1))
            def _():
                _hin(t + 1, ni_b)

            @pl.when((t & 1) == 0)
            def _():
                pltpu.async_copy(n2g_sp.at[ni_b], g_b, sem2).wait()

            @pl.when((t & 1) == 1)
            def _():
                pltpu.async_copy(n2g_sp.at[no_b], g_b, sem2).wait()

            def _hvec(i, carry2):
                g = g_b[pl.ds(i * 16, 16)]
                plsc.addupdate_scatter(ta_b, [lane64 + g], ones)
                return carry2
            lax.fori_loop(0, _K2 // 16, _hvec, 0)
            return carry
        lax.fori_loop(0, _NW2, _hwin, 0)

        for bg in range(4):
            acc = zeros
            for l in range(16):
                acc = acc + ta_b[pl.ds(l * 64 + bg * 16, 16)]
            stage[pl.ds(bg * 16, 16)] = acc
        pltpu.sync_copy(stage, hist_sp.at[pl.ds(widx * 64, 64)])

    @pl.when(wid == 0)
    def _():
        los = [zeros] * 4
        his = [jnp.full((16,), _NN, jnp.int32)] * 4

        def _bs(_, carry):
            los_, his_ = carry
            for bg in range(4):
                ni_b[pl.ds(bg * 16, 16)] = jnp.minimum(
                    (los_[bg] + his_[bg]) >> 1, _NN - 1)
            pltpu.async_copy(n2g_sp.at[ni_b.at[pl.ds(0, 64)]],
                             g_b.at[pl.ds(0, 64)], sem).wait()
            nlo, nhi = [], []
            for bg in range(4):
                b = iota + bg * 16
                lo_, hi_ = los_[bg], his_[bg]
                live = lo_ < hi_
                mid = jnp.minimum((lo_ + hi_) >> 1, _NN - 1)
                v = g_b[pl.ds(bg * 16, 16)]
                go_right = live & (v < b)
                nlo.append(jnp.where(go_right, mid + 1, lo_))
                nhi.append(jnp.where(live & jnp.logical_not(go_right),
                                     mid, hi_))
            return (nlo, nhi)
        los, his = lax.fori_loop(0, 16, _bs, (los, his))
        for bg in range(4):
            stage[pl.ds(bg * 16, 16)] = los[bg]
        pltpu.sync_copy(stage, nstart_hbm)

    plsc.subcore_barrier()
    pltpu.sync_copy(hist_sp, histv)

    tot = []
    for bg in range(4):
        def _acc(w, carry):
            t_, p_, c0_ = carry
            row = histv[pl.ds(w * 64 + bg * 16, 16)]
            return (t_ + row, p_ + jnp.where(w < wid, row, 0),
                    c0_ + jnp.where(w < _NS, row, 0))
        t_, p_, c0_ = lax.fori_loop(0, _W, _acc, (zeros, zeros, zeros))
        tot.append((t_, p_, c0_))
    carry = jnp.int32(0)
    for bg in range(4):
        t_, p_, c0_ = tot[bg]
        excl = plsc.cumsum(t_) - t_ + carry
        cur[pl.ds(bg * 16, 16)] = excl + p_
        lcnt = jnp.where(c == 1, t_ - c0_, c0_)
        gst = excl + jnp.where(c == 1, c0_, 0)
        lcnt_t[pl.ds(bg * 16, 16)] = lcnt
        gst_t[pl.ds(bg * 16, 16)] = gst
        carry = carry + jnp.sum(t_)

    lane0 = iota == 0

    def _seg(b, run):
        bb = jnp.full((16,), b, jnp.int32)
        g_ = plsc.load_gather(gst_t, [bb])[0]
        l_ = plsc.load_gather(lcnt_t, [bb])[0]
        s_ = run + ((g_ - run) & 15)
        plsc.store_scatter(s_t, [bb], jnp.full((16,), s_, jnp.int32),
                           mask=lane0)
        plsc.store_scatter(shift_t, [bb], jnp.full((16,), s_ - g_, jnp.int32),
                           mask=lane0)
        return s_ + l_
    lax.fori_loop(0, 64, _seg, jnp.int32(0))

    @pl.when(wid == 0)
    def _():
        for bg in range(4):
            stage[pl.ds(bg * 16, 16)] = tot[bg][0]
        pltpu.sync_copy(stage, nedge_hbm)

    def _win(t, carry):
        base = wid * _EPW + t * _K2
        pltpu.sync_copy(ni_hbm.at[pl.ds(base, _K2)], ni_b)
        pltpu.sync_copy(no_hbm.at[pl.ds(base, _K2)], no_b)
        pltpu.sync_copy(r_hbm.at[pl.ds(base, _K2)], r_b)
        gth = [
            pltpu.async_copy(n2g_sp.at[ni_b], g_b, sem),
            pltpu.async_copy(a2r_sp.at[ni_b], ri_b, sem),
            pltpu.async_copy(a2r_sp.at[no_b], ro_b, sem),
        ]
        for gg in gth:
            gg.wait()

        @pl.when(t > 0)
        def _():
            pltpu.make_async_copy(ta_hbm.at[pl.ds(0, _K2)], ta_b, sem2).wait()
            pltpu.make_async_copy(tb_hbm.at[pl.ds(0, _K2)], tb_b, sem2).wait()

        def _chunk(j, carry2):
            for v in range(_CH // 16):
                o16 = j * _CH + v * 16
                ni_v = ni_b[pl.ds(o16, 16)]
                no_v = no_b[pl.ds(o16, 16)]
                r_v = r_b[pl.ds(o16, 16)]
                g = g_b[pl.ds(o16, 16)]
                cnt, lastm = plsc.scan_count(g)
                basev = plsc.load_gather(cur, [g])
                dest = basev + cnt - 1
                plsc.store_scatter(cur, [g], dest + 1, mask=lastm)
                dest_b[j, pl.ds(v * 16, 16)] = dest + plsc.load_gather(shift_t, [g])
                ri = ri_b[pl.ds(o16, 16)]
                ro = ro_b[pl.ds(o16, 16)]
                sd = jnp.minimum(jnp.abs(ri - ro), _MAX_SEQ_DIST)
                ta_b[pl.ds(o16, 16)] = ni_v | lax.shift_left(no_v & 0x7FFF, 17)
                tb_b[pl.ds(o16, 16)] = (lax.shift_right_logical(no_v, 15)
                                        | lax.shift_left(r_v, 2)
                                        | lax.shift_left(sd, 5)
                                        | lax.shift_left(g, 9))
            return carry2
        lax.fori_loop(0, _K2 // _CH, _chunk, 0)

        for j in range(_K2 // _CH):
            pltpu.async_copy(ta_b.at[pl.ds(j * _CH, _CH)],
                             ta_sp.at[dest_b.at[j]], sem2)
            pltpu.async_copy(tb_b.at[pl.ds(j * _CH, _CH)],
                             tb_sp.at[dest_b.at[j]], sem2)
        return carry
    lax.fori_loop(0, _NW2, _win, 0)
    pltpu.make_async_copy(ta_hbm.at[pl.ds(0, _K2)], ta_b, sem2).wait()
    pltpu.make_async_copy(tb_hbm.at[pl.ds(0, _K2)], tb_b, sem2).wait()

    plsc.subcore_barrier()

    def _edge_scatter(sp, hbm, src, dst, lo, hi):
        pltpu.sync_copy(sp.at[pl.ds(pl.multiple_of(src, 8), 16)], tmp16)
        m = (iota >= lo) & (iota < hi)
        idx16[...] = jnp.where(m, dst + iota, _NE + iota)
        pltpu.async_copy(tmp16, hbm.at[idx16], sem).wait()

    def _drain(sp, hbm, b):
        bb = jnp.full((16,), b, jnp.int32)
        ll = plsc.load_gather(lcnt_t, [bb])[0]
        dd = plsc.load_gather(gst_t, [bb])[0]
        ss = plsc.load_gather(s_t, [bb])[0]

        @pl.when(ll > 0)
        def _():
            phase = dd & 15
            h = jnp.minimum(ll, (16 - phase) & 15)

            @pl.when(h > 0)
            def _():
                _edge_scatter(sp, hbm, ss - phase, dd - phase, phase,
                              phase + h)
            m_ = ll - h
            t15 = m_ & 15
            d0 = dd + h
            s0 = ss + h
            nfull = m_ // _DC

            @pl.when(nfull >= 1)
            def _():
                pltpu.async_copy(
                    sp.at[pl.ds(pl.multiple_of(s0, 8), _DC)], dbuf, sem)

            def _chunk(k, carry2):
                pltpu.make_async_copy(hbm.at[pl.ds(0, _DC)], dbuf, sem).wait()

                @pl.when(k >= 1)
                def _():
                    pltpu.make_async_copy(hbm.at[pl.ds(0, _DC)], dbuf,
                                          sem2).wait()

                @pl.when((k + 1 < nfull) & ((k & 1) == 1))
                def _():
                    pltpu.async_copy(
                        sp.at[pl.ds(pl.multiple_of(s0 + (k + 1) * _DC, 8),
                                    _DC)], dbuf, sem)

                @pl.when((k + 1 < nfull) & ((k & 1) == 0))
                def _():
                    pltpu.async_copy(
                        sp.at[pl.ds(pl.multiple_of(s0 + (k + 1) * _DC, 8),
                                    _DC)], dbuf2, sem)

                @pl.when((k & 1) == 0)
                def _():
                    pltpu.async_copy(
                        dbuf, hbm.at[pl.ds(pl.multiple_of(d0 + k * _DC, 8),
                                           _DC)], sem2)

                @pl.when((k & 1) == 1)
                def _():
                    pltpu.async_copy(
                        dbuf2, hbm.at[pl.ds(pl.multiple_of(d0 + k * _DC, 8),
                                            _DC)], sem2)
                return carry2
            lax.fori_loop(0, nfull, _chunk, 0)

            @pl.when(nfull >= 1)
            def _():
                pltpu.make_async_copy(hbm.at[pl.ds(0, _DC)], dbuf,
                                      sem2).wait()
            rem = (m_ - t15) - nfull * _DC
            off = nfull * _DC
            sz = _DC // 2
            while sz >= 16:
                pre = off + (rem & ~(2 * sz - 1))

                @pl.when((rem & sz) != 0)
                def _(pre=pre, sz=sz):
                    pltpu.sync_copy(
                        sp.at[pl.ds(pl.multiple_of(s0 + pre, 8), sz)],
                        dbuf.at[pl.ds(0, sz)])
                    pltpu.sync_copy(
                        dbuf.at[pl.ds(0, sz)],
                        hbm.at[pl.ds(pl.multiple_of(d0 + pre, 8), sz)])
                sz //= 2

            @pl.when(t15 > 0)
            def _():
                _edge_scatter(sp, hbm, s0 + m_ - t15, d0 + m_ - t15, 0, t15)

    def _bkt(bi, carry):
        b = s * 4 + bi
        _drain(ta_sp, ta_hbm, b)
        _drain(tb_sp, tb_hbm, b)
        return carry
    lax.fori_loop(0, 4, _bkt, 0)


def _feat_body(ta_hbm, tb_hbm, px_hbm, py_hbm, pz_hbm, nstart_hbm,
               el0_hbm, el1_hbm, el2_hbm, featf_hbm, offs_hbm,
               px_sp, py_sp, pz_sp,
               nst_v, ta_b, tb_b, ni_b, no_b,
               pxi, pyi, pzi, pxo, pyo, pzo,
               c1_b, c2_b, e0_b, e1_b, e2_b, feat_b, offs_b, sem, sem2):
    c = lax.axis_index("c")
    s = lax.axis_index("s")
    wid = c * _NS + s

    @pl.when(s == 0)
    def _():
        pltpu.sync_copy(px_hbm, px_sp)
        pltpu.sync_copy(py_hbm, py_sp)
        pltpu.sync_copy(pz_hbm, pz_sp)
    plsc.subcore_barrier()

    pltpu.sync_copy(nstart_hbm, nst_v)
    iota = lax.iota(jnp.int32, 16)
    zf = jnp.zeros((16,), jnp.float32)
    onesf = jnp.ones((16,), jnp.float32)

    def _zf(i, carry):
        feat_b[pl.ds(i * 16, 16)] = zf
        return carry
    lax.fori_loop(0, _K2 * _FDIM // 16, _zf, 0)

    def _win(t, carry):
        base = wid * _EPW + t * _K2
        pltpu.sync_copy(ta_hbm.at[pl.ds(base, _K2)], ta_b)
        pltpu.sync_copy(tb_hbm.at[pl.ds(base, _K2)], tb_b)

        def _unp(i, carry2):
            o16 = i * 16
            ta_v = ta_b[pl.ds(o16, 16)]
            tb_v = tb_b[pl.ds(o16, 16)]
            ni_v = ta_v & 0x1FFFF
            no_v = (lax.shift_right_logical(ta_v, 17)
                    | lax.shift_left(tb_v & 0x3, 15))
            ni_b[pl.ds(o16, 16)] = ni_v
            no_b[pl.ds(o16, 16)] = no_v
            return carry2
        lax.fori_loop(0, _K2 // 16, _unp, 0)

        gth = [
            pltpu.async_copy(px_sp.at[ni_b], pxi, sem),
            pltpu.async_copy(py_sp.at[ni_b], pyi, sem),
            pltpu.async_copy(pz_sp.at[ni_b], pzi, sem),
            pltpu.async_copy(px_sp.at[no_b], pxo, sem),
            pltpu.async_copy(py_sp.at[no_b], pyo, sem),
            pltpu.async_copy(pz_sp.at[no_b], pzo, sem),
        ]
        for gg in gth:
            gg.wait()

        @pl.when(t > 0)
        def _():
            pltpu.make_async_copy(el0_hbm.at[pl.ds(0, _K2)], e0_b,
                                  sem2).wait()
            pltpu.make_async_copy(el1_hbm.at[pl.ds(0, _K2)], e1_b,
                                  sem2).wait()
            pltpu.make_async_copy(el2_hbm.at[pl.ds(0, _K2)], e2_b,
                                  sem2).wait()
            pltpu.make_async_copy(featf_hbm.at[pl.ds(0, _FDIM * _K2)],
                                  feat_b, sem2).wait()
            pltpu.make_async_copy(offs_hbm.at[pl.ds(0, _K2)], offs_b,
                                  sem2).wait()

            def _rz(i, carry2):
                cr = c1_b[pl.ds(i * 16, 16)]
                cs = c2_b[pl.ds(i * 16, 16)]
                plsc.store_scatter(feat_b, [cr], zf)
                plsc.store_scatter(feat_b, [cs], zf)
                return carry2
            lax.fori_loop(0, _K2 // 16, _rz, 0)

        def _vec(i, carry2):
            o16 = i * 16
            tb_v = tb_b[pl.ds(o16, 16)]
            ni_v = ni_b[pl.ds(o16, 16)]
            no_v = no_b[pl.ds(o16, 16)]
            r_v = lax.shift_right_logical(tb_v, 2) & 0x7
            sd = lax.shift_right_logical(tb_v, 5) & 0xF
            g = lax.shift_right_logical(tb_v, 9)
            offs_b[pl.ds(o16, 16)] = plsc.load_gather(nst_v, [g])
            e0_b[pl.ds(o16, 16)] = ni_v
            e1_b[pl.ds(o16, 16)] = no_v
            e2_b[pl.ds(o16, 16)] = r_v
            lane25 = (iota + o16) * _FDIM
            cr = lane25 + 6 + r_v
            cs = lane25 + 13 + sd
            c1_b[pl.ds(o16, 16)] = cr
            c2_b[pl.ds(o16, 16)] = cs
            plsc.store_scatter(feat_b, [cr], onesf)
            plsc.store_scatter(feat_b, [cs], onesf)
            xi = pxi[pl.ds(o16, 16)]
            yi = pyi[pl.ds(o16, 16)]
            zi = pzi[pl.ds(o16, 16)]
            xo = pxo[pl.ds(o16, 16)]
            yo = pyo[pl.ds(o16, 16)]
            zo = pzo[pl.ds(o16, 16)]
            plsc.store_scatter(feat_b, [lane25], xi)
            plsc.store_scatter(feat_b, [lane25 + 1], yi)
            plsc.store_scatter(feat_b, [lane25 + 2], zi)
            plsc.store_scatter(feat_b, [lane25 + 3], xo)
            plsc.store_scatter(feat_b, [lane25 + 4], yo)
            plsc.store_scatter(feat_b, [lane25 + 5], zo)
            dx = xi - xo
            dy = yi - yo
            dz = zi - zo
            d2 = dx * dx + dy * dy + dz * dz + jnp.float32(1e-12)
            ibits = plsc.bitcast(d2, jnp.int32)
            y0 = plsc.bitcast(_RSQRT_MAGIC - lax.shift_right_logical(ibits, 1),
                              jnp.float32)
            for _ in range(3):
                y0 = y0 * (jnp.float32(1.5) - jnp.float32(0.5) * d2 * y0 * y0)
            dist = d2 * y0
            plsc.store_scatter(feat_b, [lane25 + 24], dist)
            return carry2
        lax.fori_loop(0, _K2 // 16, _vec, 0)

        pltpu.async_copy(e0_b, el0_hbm.at[pl.ds(base, _K2)], sem2)
        pltpu.async_copy(e1_b, el1_hbm.at[pl.ds(base, _K2)], sem2)
        pltpu.async_copy(e2_b, el2_hbm.at[pl.ds(base, _K2)], sem2)
        pltpu.async_copy(feat_b,
                         featf_hbm.at[pl.ds(_FDIM * base, _FDIM * _K2)],
                         sem2)
        pltpu.async_copy(offs_b, offs_hbm.at[pl.ds(base, _K2)], sem2)
        return carry
    lax.fori_loop(0, _NW2, _win, 0)
    pltpu.make_async_copy(el0_hbm.at[pl.ds(0, _K2)], e0_b, sem2).wait()
    pltpu.make_async_copy(el1_hbm.at[pl.ds(0, _K2)], e1_b, sem2).wait()
    pltpu.make_async_copy(el2_hbm.at[pl.ds(0, _K2)], e2_b, sem2).wait()
    pltpu.make_async_copy(featf_hbm.at[pl.ds(0, _FDIM * _K2)], feat_b,
                          sem2).wait()
    pltpu.make_async_copy(offs_hbm.at[pl.ds(0, _K2)], offs_b, sem2).wait()


def _build_rank():
    return pl.kernel(
        _rank_body,
        out_type=(
            jax.ShapeDtypeStruct((_NE + 16,), jnp.int32),
            jax.ShapeDtypeStruct((_NE + 16,), jnp.int32),
            jax.ShapeDtypeStruct((64,), jnp.int32),
            jax.ShapeDtypeStruct((64,), jnp.int32),
        ),
        mesh=_mesh(),
        compiler_params=_PARAMS,
        scratch_types=[
            pltpu.VMEM_SHARED((_SPC,), jnp.int32),
            pltpu.VMEM_SHARED((_SPC,), jnp.int32),
            pltpu.VMEM_SHARED((_NN,), jnp.int32),
            pltpu.VMEM_SHARED((_NN,), jnp.int32),
            pltpu.VMEM_SHARED((_W * 64,), jnp.int32),
            pltpu.VMEM((_W * 64,), jnp.int32),
            pltpu.VMEM((64,), jnp.int32),
            pltpu.VMEM((64,), jnp.int32),

            pltpu.VMEM((80,), jnp.int32),
            pltpu.VMEM((80,), jnp.int32),
            pltpu.VMEM((80,), jnp.int32),
            pltpu.VMEM((80,), jnp.int32),
            pltpu.VMEM((16,), jnp.int32),
            pltpu.VMEM((16,), jnp.int32),
            pltpu.VMEM((_DC,), jnp.int32),
            pltpu.VMEM((_DC,), jnp.int32),
            pltpu.VMEM((_K2,), jnp.int32),
            pltpu.VMEM((_K2,), jnp.int32),
            pltpu.VMEM((_K2,), jnp.int32),
            pltpu.VMEM((_K2,), jnp.int32),
            pltpu.VMEM((_K2,), jnp.int32),
            pltpu.VMEM((_K2,), jnp.int32),
            pltpu.VMEM((_K2,), jnp.int32),
            pltpu.VMEM((_K2,), jnp.int32),
            pltpu.VMEM((_K2 // _CH, _CH), jnp.int32),
            pltpu.SemaphoreType.DMA,
            pltpu.SemaphoreType.DMA,
        ],
    )


def _build_feat():
    return pl.kernel(
        _feat_body,
        out_type=(
            jax.ShapeDtypeStruct((_NE,), jnp.int32),
            jax.ShapeDtypeStruct((_NE,), jnp.int32),
            jax.ShapeDtypeStruct((_NE,), jnp.int32),
            jax.ShapeDtypeStruct((_FDIM * _NE,), jnp.float32),
            jax.ShapeDtypeStruct((_NE,), jnp.int32),
        ),
        mesh=_mesh(),
        compiler_params=_PARAMS,
        scratch_types=[
            pltpu.VMEM_SHARED((_NN,), jnp.float32),
            pltpu.VMEM_SHARED((_NN,), jnp.float32),
            pltpu.VMEM_SHARED((_NN,), jnp.float32),
            pltpu.VMEM((64,), jnp.int32),
            pltpu.VMEM((_K2,), jnp.int32),
            pltpu.VMEM((_K2,), jnp.int32),
            pltpu.VMEM((_K2,), jnp.int32),
            pltpu.VMEM((_K2,), jnp.int32),
            pltpu.VMEM((_K2,), jnp.float32),
            pltpu.VMEM((_K2,), jnp.float32),
            pltpu.VMEM((_K2,), jnp.float32),
            pltpu.VMEM((_K2,), jnp.float32),
            pltpu.VMEM((_K2,), jnp.float32),
            pltpu.VMEM((_K2,), jnp.float32),
            pltpu.VMEM((_K2,), jnp.int32),
            pltpu.VMEM((_K2,), jnp.int32),
            pltpu.VMEM((_K2,), jnp.int32),
            pltpu.VMEM((_K2,), jnp.int32),
            pltpu.VMEM((_K2,), jnp.int32),
            pltpu.VMEM((_FDIM * _K2,), jnp.float32),
            pltpu.VMEM((_K2,), jnp.int32),
            pltpu.SemaphoreType.DMA,
            pltpu.SemaphoreType.DMA,
        ],
    )


def kernel(node_in, node_out, relation, node_position, atom2residue,
           residue_type, node2graph):
    del residue_type
    px = node_position[:, 0]
    py = node_position[:, 1]
    pz = node_position[:, 2]
    ta, tb, num_edges, nstart = _build_rank()(node_in, node_out, relation,
                                              atom2residue, node2graph)
    el0, el1, el2, featf, offsets = _build_feat()(ta, tb, px, py, pz, nstart)
    edge_list = jnp.stack([el0, el1, el2], axis=1)
    edge_feature = featf.reshape(_NE, _FDIM)
    return edge_list, edge_feature, num_edges, offsets


# speedup vs baseline: 1.0239x; 1.0239x over previous
import jax
import jax.numpy as jnp
from jax import lax
from jax.experimental import pallas as pl
from jax.experimental.pallas import tpu as pltpu
from jax.experimental.pallas import tpu_sc as plsc

_NUM_RELATION = 7
_MAX_SEQ_DIST = 10
_B = 64
_NN = 50000
_NE = 1600000
_NC = 2
_NS = 16
_W = _NC * _NS
_EPW = _NE // _W
_K1 = 2000
_NW1 = _EPW // _K1
_K2 = 2000
_NW2 = _EPW // _K2
_FDIM = 25
_RSQRT_MAGIC = 0x5F3759DF
_CH = 80
_DC = 256
_SPC = _NS * _EPW + 64 * 16 + 32

_PARAMS = pltpu.CompilerParams(needs_layout_passes=False)


def _mesh():
    return plsc.VectorSubcoreMesh(
        core_axis_name="c", subcore_axis_name="s",
        num_cores=_NC, num_subcores=_NS)


def _rank_body(ni_hbm, no_hbm, r_hbm, a2r_hbm, n2g_hbm,
               ta_hbm, tb_hbm, nedge_hbm, nstart_hbm,
               ta_sp, tb_sp, n2g_sp, a2r_sp, hist_sp,
               histv, cur, stage,
               lcnt_t, gst_t, s_t, shift_t, tmp16, idx16, dbuf, dbuf2,
               ni_b, no_b, r_b, ta_b, tb_b, g_b, ri_b, ro_b, dest_b,
               sem, sem2):
    c = lax.axis_index("c")
    s = lax.axis_index("s")
    wid = c * _NS + s

    @pl.when(s == 0)
    def _():
        pltpu.sync_copy(n2g_hbm, n2g_sp)
        pltpu.sync_copy(a2r_hbm, a2r_sp)
    plsc.subcore_barrier()

    iota = lax.iota(jnp.int32, 16)
    zeros = jnp.zeros((16,), jnp.int32)
    ones = jnp.ones((16,), jnp.int32)
    lane64 = iota * 64

    for widx_off in (0, _NS):
        widx = s + widx_off

        def _z(i, carry):
            ta_b[pl.ds(i * 16, 16)] = zeros
            return carry
        lax.fori_loop(0, 64, _z, 0)

        def _hin(t, buf):
            pltpu.async_copy(ni_hbm.at[pl.ds(widx * _EPW + t * _K2, _K2)],
                             buf, sem)

        _hin(0, ni_b)

        def _hwin(t, carry):
            pltpu.make_async_copy(ni_hbm.at[pl.ds(0, _K2)], ni_b, sem).wait()

            @pl.when(((t + 1) < _NW2) & ((t & 1) == 0))
            def _():
                _hin(t + 1, no_b)

            @pl.when(((t + 1) < _NW2) & ((t & 1) == 1))
            def _():
                _hin(t + 1, ni_b)

            @pl.when((t & 1) == 0)
            def _():
                pltpu.async_copy(n2g_sp.at[ni_b], g_b, sem2).wait()

            @pl.when((t & 1) == 1)
            def _():
                pltpu.async_copy(n2g_sp.at[no_b], g_b, sem2).wait()

            def _hvec(i, carry2):
                g = g_b[pl.ds(i * 16, 16)]
                plsc.addupdate_scatter(ta_b, [lane64 + g], ones)
                return carry2
            lax.fori_loop(0, _K2 // 16, _hvec, 0)
            return carry
        lax.fori_loop(0, _NW2, _hwin, 0)

        for bg in range(4):
            acc = zeros
            for l in range(16):
                acc = acc + ta_b[pl.ds(l * 64 + bg * 16, 16)]
            stage[pl.ds(bg * 16, 16)] = acc
        pltpu.sync_copy(stage, hist_sp.at[pl.ds(widx * 64, 64)])

    @pl.when(wid == 0)
    def _():
        los = [zeros] * 4
        his = [jnp.full((16,), _NN, jnp.int32)] * 4

        def _bs(_, carry):
            los_, his_ = carry
            for bg in range(4):
                ni_b[pl.ds(bg * 16, 16)] = jnp.minimum(
                    (los_[bg] + his_[bg]) >> 1, _NN - 1)
            pltpu.async_copy(n2g_sp.at[ni_b.at[pl.ds(0, 64)]],
                             g_b.at[pl.ds(0, 64)], sem).wait()
            nlo, nhi = [], []
            for bg in range(4):
                b = iota + bg * 16
                lo_, hi_ = los_[bg], his_[bg]
                live = lo_ < hi_
                mid = jnp.minimum((lo_ + hi_) >> 1, _NN - 1)
                v = g_b[pl.ds(bg * 16, 16)]
                go_right = live & (v < b)
                nlo.append(jnp.where(go_right, mid + 1, lo_))
                nhi.append(jnp.where(live & jnp.logical_not(go_right),
                                     mid, hi_))
            return (nlo, nhi)
        los, his = lax.fori_loop(0, 16, _bs, (los, his))
        for bg in range(4):
            stage[pl.ds(bg * 16, 16)] = los[bg]
        pltpu.sync_copy(stage, nstart_hbm)

    plsc.subcore_barrier()
    pltpu.sync_copy(hist_sp, histv)

    tot = []
    for bg in range(4):
        def _acc(w, carry):
            t_, p_, c0_ = carry
            row = histv[pl.ds(w * 64 + bg * 16, 16)]
            return (t_ + row, p_ + jnp.where(w < wid, row, 0),
                    c0_ + jnp.where(w < _NS, row, 0))
        t_, p_, c0_ = lax.fori_loop(0, _W, _acc, (zeros, zeros, zeros))
        tot.append((t_, p_, c0_))
    carry = jnp.int32(0)
    for bg in range(4):
        t_, p_, c0_ = tot[bg]
        excl = plsc.cumsum(t_) - t_ + carry
        cur[pl.ds(bg * 16, 16)] = excl + p_
        lcnt = jnp.where(c == 1, t_ - c0_, c0_)
        gst = excl + jnp.where(c == 1, c0_, 0)
        lcnt_t[pl.ds(bg * 16, 16)] = lcnt
        gst_t[pl.ds(bg * 16, 16)] = gst
        carry = carry + jnp.sum(t_)

    lane0 = iota == 0

    def _seg(b, run):
        bb = jnp.full((16,), b, jnp.int32)
        g_ = plsc.load_gather(gst_t, [bb])[0]
        l_ = plsc.load_gather(lcnt_t, [bb])[0]
        s_ = run + ((g_ - run) & 15)
        plsc.store_scatter(s_t, [bb], jnp.full((16,), s_, jnp.int32),
                           mask=lane0)
        plsc.store_scatter(shift_t, [bb], jnp.full((16,), s_ - g_, jnp.int32),
                           mask=lane0)
        return s_ + l_
    lax.fori_loop(0, 64, _seg, jnp.int32(0))

    @pl.when(wid == 0)
    def _():
        for bg in range(4):
            stage[pl.ds(bg * 16, 16)] = tot[bg][0]
        pltpu.sync_copy(stage, nedge_hbm)

    def _win(t, carry):
        base = wid * _EPW + t * _K2
        ins = [
            pltpu.async_copy(ni_hbm.at[pl.ds(base, _K2)], ni_b, sem),
            pltpu.async_copy(no_hbm.at[pl.ds(base, _K2)], no_b, sem),
            pltpu.async_copy(r_hbm.at[pl.ds(base, _K2)], r_b, sem),
        ]
        for ic in ins:
            ic.wait()
        gth = [
            pltpu.async_copy(n2g_sp.at[ni_b], g_b, sem),
            pltpu.async_copy(a2r_sp.at[ni_b], ri_b, sem),
            pltpu.async_copy(a2r_sp.at[no_b], ro_b, sem),
        ]
        for gg in gth:
            gg.wait()

        @pl.when(t > 0)
        def _():
            pltpu.make_async_copy(ta_hbm.at[pl.ds(0, _K2)], ta_b, sem2).wait()
            pltpu.make_async_copy(tb_hbm.at[pl.ds(0, _K2)], tb_b, sem2).wait()

        def _chunk(j, carry2):
            for v in range(_CH // 16):
                o16 = j * _CH + v * 16
                ni_v = ni_b[pl.ds(o16, 16)]
                no_v = no_b[pl.ds(o16, 16)]
                r_v = r_b[pl.ds(o16, 16)]
                g = g_b[pl.ds(o16, 16)]
                cnt, lastm = plsc.scan_count(g)
                basev = plsc.load_gather(cur, [g])
                dest = basev + cnt - 1
                plsc.store_scatter(cur, [g], dest + 1, mask=lastm)
                dest_b[j, pl.ds(v * 16, 16)] = dest + plsc.load_gather(shift_t, [g])
                ri = ri_b[pl.ds(o16, 16)]
                ro = ro_b[pl.ds(o16, 16)]
                sd = jnp.minimum(jnp.abs(ri - ro), _MAX_SEQ_DIST)
                ta_b[pl.ds(o16, 16)] = ni_v | lax.shift_left(no_v & 0x7FFF, 17)
                tb_b[pl.ds(o16, 16)] = (lax.shift_right_logical(no_v, 15)
                                        | lax.shift_left(r_v, 2)
                                        | lax.shift_left(sd, 5)
                                        | lax.shift_left(g, 9))
            return carry2
        lax.fori_loop(0, _K2 // _CH, _chunk, 0)

        for j in range(_K2 // _CH):
            pltpu.async_copy(ta_b.at[pl.ds(j * _CH, _CH)],
                             ta_sp.at[dest_b.at[j]], sem2)
            pltpu.async_copy(tb_b.at[pl.ds(j * _CH, _CH)],
                             tb_sp.at[dest_b.at[j]], sem2)
        return carry
    lax.fori_loop(0, _NW2, _win, 0)
    pltpu.make_async_copy(ta_hbm.at[pl.ds(0, _K2)], ta_b, sem2).wait()
    pltpu.make_async_copy(tb_hbm.at[pl.ds(0, _K2)], tb_b, sem2).wait()

    plsc.subcore_barrier()

    def _edge_scatter(sp, hbm, src, dst, lo, hi):
        pltpu.sync_copy(sp.at[pl.ds(pl.multiple_of(src, 8), 16)], tmp16)
        m = (iota >= lo) & (iota < hi)
        idx16[...] = jnp.where(m, dst + iota, _NE + iota)
        pltpu.async_copy(tmp16, hbm.at[idx16], sem).wait()

    def _drain(sp, hbm, b):
        bb = jnp.full((16,), b, jnp.int32)
        ll = plsc.load_gather(lcnt_t, [bb])[0]
        dd = plsc.load_gather(gst_t, [bb])[0]
        ss = plsc.load_gather(s_t, [bb])[0]

        @pl.when(ll > 0)
        def _():
            phase = dd & 15
            h = jnp.minimum(ll, (16 - phase) & 15)

            @pl.when(h > 0)
            def _():
                _edge_scatter(sp, hbm, ss - phase, dd - phase, phase,
                              phase + h)
            m_ = ll - h
            t15 = m_ & 15
            d0 = dd + h
            s0 = ss + h
            nfull = m_ // _DC

            @pl.when(nfull >= 1)
            def _():
                pltpu.async_copy(
                    sp.at[pl.ds(pl.multiple_of(s0, 8), _DC)], dbuf, sem)

            def _chunk(k, carry2):
                pltpu.make_async_copy(hbm.at[pl.ds(0, _DC)], dbuf, sem).wait()

                @pl.when(k >= 1)
                def _():
                    pltpu.make_async_copy(hbm.at[pl.ds(0, _DC)], dbuf,
                                          sem2).wait()

                @pl.when((k + 1 < nfull) & ((k & 1) == 1))
                def _():
                    pltpu.async_copy(
                        sp.at[pl.ds(pl.multiple_of(s0 + (k + 1) * _DC, 8),
                                    _DC)], dbuf, sem)

                @pl.when((k + 1 < nfull) & ((k & 1) == 0))
                def _():
                    pltpu.async_copy(
                        sp.at[pl.ds(pl.multiple_of(s0 + (k + 1) * _DC, 8),
                                    _DC)], dbuf2, sem)

                @pl.when((k & 1) == 0)
                def _():
                    pltpu.async_copy(
                        dbuf, hbm.at[pl.ds(pl.multiple_of(d0 + k * _DC, 8),
                                           _DC)], sem2)

                @pl.when((k & 1) == 1)
                def _():
                    pltpu.async_copy(
                        dbuf2, hbm.at[pl.ds(pl.multiple_of(d0 + k * _DC, 8),
                                            _DC)], sem2)
                return carry2
            lax.fori_loop(0, nfull, _chunk, 0)

            @pl.when(nfull >= 1)
            def _():
                pltpu.make_async_copy(hbm.at[pl.ds(0, _DC)], dbuf,
                                      sem2).wait()
            rem = (m_ - t15) - nfull * _DC
            off = nfull * _DC
            sz = _DC // 2
            while sz >= 16:
                pre = off + (rem & ~(2 * sz - 1))

                @pl.when((rem & sz) != 0)
                def _(pre=pre, sz=sz):
                    pltpu.sync_copy(
                        sp.at[pl.ds(pl.multiple_of(s0 + pre, 8), sz)],
                        dbuf.at[pl.ds(0, sz)])
                    pltpu.sync_copy(
                        dbuf.at[pl.ds(0, sz)],
                        hbm.at[pl.ds(pl.multiple_of(d0 + pre, 8), sz)])
                sz //= 2

            @pl.when(t15 > 0)
            def _():
                _edge_scatter(sp, hbm, s0 + m_ - t15, d0 + m_ - t15, 0, t15)

    def _bkt(bi, carry):
        b = s * 4 + bi
        _drain(ta_sp, ta_hbm, b)
        _drain(tb_sp, tb_hbm, b)
        return carry
    lax.fori_loop(0, 4, _bkt, 0)


def _feat_body(ta_hbm, tb_hbm, px_hbm, py_hbm, pz_hbm, nstart_hbm,
               el0_hbm, el1_hbm, el2_hbm, featf_hbm, offs_hbm,
               px_sp, py_sp, pz_sp,
               nst_v, ta_b, tb_b, ni_b, no_b,
               pxi, pyi, pzi, pxo, pyo, pzo,
               c1_b, c2_b, e0_b, e1_b, e2_b, feat_b, offs_b, sem, sem2):
    c = lax.axis_index("c")
    s = lax.axis_index("s")
    wid = c * _NS + s

    @pl.when(s == 0)
    def _():
        pltpu.sync_copy(px_hbm, px_sp)
        pltpu.sync_copy(py_hbm, py_sp)
        pltpu.sync_copy(pz_hbm, pz_sp)
    plsc.subcore_barrier()

    pltpu.sync_copy(nstart_hbm, nst_v)
    iota = lax.iota(jnp.int32, 16)
    zf = jnp.zeros((16,), jnp.float32)
    onesf = jnp.ones((16,), jnp.float32)

    def _zf(i, carry):
        feat_b[pl.ds(i * 16, 16)] = zf
        return carry
    lax.fori_loop(0, _K2 * _FDIM // 16, _zf, 0)

    def _win(t, carry):
        base = wid * _EPW + t * _K2
        ins = [
            pltpu.async_copy(ta_hbm.at[pl.ds(base, _K2)], ta_b, sem),
            pltpu.async_copy(tb_hbm.at[pl.ds(base, _K2)], tb_b, sem),
        ]
        for ic in ins:
            ic.wait()

        def _unp(i, carry2):
            o16 = i * 16
            ta_v = ta_b[pl.ds(o16, 16)]
            tb_v = tb_b[pl.ds(o16, 16)]
            ni_v = ta_v & 0x1FFFF
            no_v = (lax.shift_right_logical(ta_v, 17)
                    | lax.shift_left(tb_v & 0x3, 15))
            ni_b[pl.ds(o16, 16)] = ni_v
            no_b[pl.ds(o16, 16)] = no_v
            return carry2
        lax.fori_loop(0, _K2 // 16, _unp, 0)

        gth = [
            pltpu.async_copy(px_sp.at[ni_b], pxi, sem),
            pltpu.async_copy(py_sp.at[ni_b], pyi, sem),
            pltpu.async_copy(pz_sp.at[ni_b], pzi, sem),
            pltpu.async_copy(px_sp.at[no_b], pxo, sem),
            pltpu.async_copy(py_sp.at[no_b], pyo, sem),
            pltpu.async_copy(pz_sp.at[no_b], pzo, sem),
        ]
        for gg in gth:
            gg.wait()

        @pl.when(t > 0)
        def _():
            pltpu.make_async_copy(el0_hbm.at[pl.ds(0, _K2)], e0_b,
                                  sem2).wait()
            pltpu.make_async_copy(el1_hbm.at[pl.ds(0, _K2)], e1_b,
                                  sem2).wait()
            pltpu.make_async_copy(el2_hbm.at[pl.ds(0, _K2)], e2_b,
                                  sem2).wait()
            pltpu.make_async_copy(featf_hbm.at[pl.ds(0, _FDIM * _K2)],
                                  feat_b, sem2).wait()
            pltpu.make_async_copy(offs_hbm.at[pl.ds(0, _K2)], offs_b,
                                  sem2).wait()

            def _rz(i, carry2):
                cr = c1_b[pl.ds(i * 16, 16)]
                cs = c2_b[pl.ds(i * 16, 16)]
                plsc.store_scatter(feat_b, [cr], zf)
                plsc.store_scatter(feat_b, [cs], zf)
                return carry2
            lax.fori_loop(0, _K2 // 16, _rz, 0)

        def _vec(i, carry2):
            o16 = i * 16
            tb_v = tb_b[pl.ds(o16, 16)]
            ni_v = ni_b[pl.ds(o16, 16)]
            no_v = no_b[pl.ds(o16, 16)]
            r_v = lax.shift_right_logical(tb_v, 2) & 0x7
            sd = lax.shift_right_logical(tb_v, 5) & 0xF
            g = lax.shift_right_logical(tb_v, 9)
            offs_b[pl.ds(o16, 16)] = plsc.load_gather(nst_v, [g])
            e0_b[pl.ds(o16, 16)] = ni_v
            e1_b[pl.ds(o16, 16)] = no_v
            e2_b[pl.ds(o16, 16)] = r_v
            lane25 = (iota + o16) * _FDIM
            cr = lane25 + 6 + r_v
            cs = lane25 + 13 + sd
            c1_b[pl.ds(o16, 16)] = cr
            c2_b[pl.ds(o16, 16)] = cs
            plsc.store_scatter(feat_b, [cr], onesf)
            plsc.store_scatter(feat_b, [cs], onesf)
            xi = pxi[pl.ds(o16, 16)]
            yi = pyi[pl.ds(o16, 16)]
            zi = pzi[pl.ds(o16, 16)]
            xo = pxo[pl.ds(o16, 16)]
            yo = pyo[pl.ds(o16, 16)]
            zo = pzo[pl.ds(o16, 16)]
            plsc.store_scatter(feat_b, [lane25], xi)
            plsc.store_scatter(feat_b, [lane25 + 1], yi)
            plsc.store_scatter(feat_b, [lane25 + 2], zi)
            plsc.store_scatter(feat_b, [lane25 + 3], xo)
            plsc.store_scatter(feat_b, [lane25 + 4], yo)
            plsc.store_scatter(feat_b, [lane25 + 5], zo)
            dx = xi - xo
            dy = yi - yo
            dz = zi - zo
            d2 = dx * dx + dy * dy + dz * dz + jnp.float32(1e-12)
            ibits = plsc.bitcast(d2, jnp.int32)
            y0 = plsc.bitcast(_RSQRT_MAGIC - lax.shift_right_logical(ibits, 1),
                              jnp.float32)
            for _ in range(3):
                y0 = y0 * (jnp.float32(1.5) - jnp.float32(0.5) * d2 * y0 * y0)
            dist = d2 * y0
            plsc.store_scatter(feat_b, [lane25 + 24], dist)
            return carry2
        lax.fori_loop(0, _K2 // 16, _vec, 0)

        pltpu.async_copy(e0_b, el0_hbm.at[pl.ds(base, _K2)], sem2)
        pltpu.async_copy(e1_b, el1_hbm.at[pl.ds(base, _K2)], sem2)
        pltpu.async_copy(e2_b, el2_hbm.at[pl.ds(base, _K2)], sem2)
        pltpu.async_copy(feat_b,
                         featf_hbm.at[pl.ds(_FDIM * base, _FDIM * _K2)],
                         sem2)
        pltpu.async_copy(offs_b, offs_hbm.at[pl.ds(base, _K2)], sem2)
        return carry
    lax.fori_loop(0, _NW2, _win, 0)
    pltpu.make_async_copy(el0_hbm.at[pl.ds(0, _K2)], e0_b, sem2).wait()
    pltpu.make_async_copy(el1_hbm.at[pl.ds(0, _K2)], e1_b, sem2).wait()
    pltpu.make_async_copy(el2_hbm.at[pl.ds(0, _K2)], e2_b, sem2).wait()
    pltpu.make_async_copy(featf_hbm.at[pl.ds(0, _FDIM * _K2)], feat_b,
                          sem2).wait()
    pltpu.make_async_copy(offs_hbm.at[pl.ds(0, _K2)], offs_b, sem2).wait()


def _build_rank():
    return pl.kernel(
        _rank_body,
        out_type=(
            jax.ShapeDtypeStruct((_NE + 16,), jnp.int32),
            jax.ShapeDtypeStruct((_NE + 16,), jnp.int32),
            jax.ShapeDtypeStruct((64,), jnp.int32),
            jax.ShapeDtypeStruct((64,), jnp.int32),
        ),
        mesh=_mesh(),
        compiler_params=_PARAMS,
        scratch_types=[
            pltpu.VMEM_SHARED((_SPC,), jnp.int32),
            pltpu.VMEM_SHARED((_SPC,), jnp.int32),
            pltpu.VMEM_SHARED((_NN,), jnp.int32),
            pltpu.VMEM_SHARED((_NN,), jnp.int32),
            pltpu.VMEM_SHARED((_W * 64,), jnp.int32),
            pltpu.VMEM((_W * 64,), jnp.int32),
            pltpu.VMEM((64,), jnp.int32),
            pltpu.VMEM((64,), jnp.int32),

            pltpu.VMEM((80,), jnp.int32),
            pltpu.VMEM((80,), jnp.int32),
            pltpu.VMEM((80,), jnp.int32),
            pltpu.VMEM((80,), jnp.int32),
            pltpu.VMEM((16,), jnp.int32),
            pltpu.VMEM((16,), jnp.int32),
            pltpu.VMEM((_DC,), jnp.int32),
            pltpu.VMEM((_DC,), jnp.int32),
            pltpu.VMEM((_K2,), jnp.int32),
            pltpu.VMEM((_K2,), jnp.int32),
            pltpu.VMEM((_K2,), jnp.int32),
            pltpu.VMEM((_K2,), jnp.int32),
            pltpu.VMEM((_K2,), jnp.int32),
            pltpu.VMEM((_K2,), jnp.int32),
            pltpu.VMEM((_K2,), jnp.int32),
            pltpu.VMEM((_K2,), jnp.int32),
            pltpu.VMEM((_K2 // _CH, _CH), jnp.int32),
            pltpu.SemaphoreType.DMA,
            pltpu.SemaphoreType.DMA,
        ],
    )


def _build_feat():
    return pl.kernel(
        _feat_body,
        out_type=(
            jax.ShapeDtypeStruct((_NE,), jnp.int32),
            jax.ShapeDtypeStruct((_NE,), jnp.int32),
            jax.ShapeDtypeStruct((_NE,), jnp.int32),
            jax.ShapeDtypeStruct((_FDIM * _NE,), jnp.float32),
            jax.ShapeDtypeStruct((_NE,), jnp.int32),
        ),
        mesh=_mesh(),
        compiler_params=_PARAMS,
        scratch_types=[
            pltpu.VMEM_SHARED((_NN,), jnp.float32),
            pltpu.VMEM_SHARED((_NN,), jnp.float32),
            pltpu.VMEM_SHARED((_NN,), jnp.float32),
            pltpu.VMEM((64,), jnp.int32),
            pltpu.VMEM((_K2,), jnp.int32),
            pltpu.VMEM((_K2,), jnp.int32),
            pltpu.VMEM((_K2,), jnp.int32),
            pltpu.VMEM((_K2,), jnp.int32),
            pltpu.VMEM((_K2,), jnp.float32),
            pltpu.VMEM((_K2,), jnp.float32),
            pltpu.VMEM((_K2,), jnp.float32),
            pltpu.VMEM((_K2,), jnp.float32),
            pltpu.VMEM((_K2,), jnp.float32),
            pltpu.VMEM((_K2,), jnp.float32),
            pltpu.VMEM((_K2,), jnp.int32),
            pltpu.VMEM((_K2,), jnp.int32),
            pltpu.VMEM((_K2,), jnp.int32),
            pltpu.VMEM((_K2,), jnp.int32),
            pltpu.VMEM((_K2,), jnp.int32),
            pltpu.VMEM((_FDIM * _K2,), jnp.float32),
            pltpu.VMEM((_K2,), jnp.int32),
            pltpu.SemaphoreType.DMA,
            pltpu.SemaphoreType.DMA,
        ],
    )


def kernel(node_in, node_out, relation, node_position, atom2residue,
           residue_type, node2graph):
    del residue_type
    px = node_position[:, 0]
    py = node_position[:, 1]
    pz = node_position[:, 2]
    ta, tb, num_edges, nstart = _build_rank()(node_in, node_out, relation,
                                              atom2residue, node2graph)
    el0, el1, el2, featf, offsets = _build_feat()(ta, tb, px, py, pz, nstart)
    edge_list = jnp.stack([el0, el1, el2], axis=1)
    edge_feature = featf.reshape(_NE, _FDIM)
    return edge_list, edge_feature, num_edges, offsets
